# baseline (speedup 1.0000x reference)
import functools

import jax
import jax.numpy as jnp
from jax import lax
from jax.experimental import pallas as pl
from jax.experimental.pallas import tpu as pltpu
from jax.experimental.pallas import tpu_sc as plsc

N_NODES = 100000
K = 10
DIM_OUT = 10
GW = 16
N_EDGES = 6400000
EPO = 1600000
N_ORDERS = 4

NC, NS = 2, 16
ROW = 512
NROW = EPO // ROW
NW = NC * NS
W_HI = NROW - (NROW // NW) * NW
N_PAD = 100096
RPT = N_PAD // NS
BN = 3128


_PREC = lax.Precision.HIGHEST


def _dot(a, b):
  return jnp.dot(a, b, preferred_element_type=jnp.float32, precision=_PREC)


def _build_body(h_ref, wb_ref, g_ref):
  g_ref[...] = _dot(h_ref[...], wb_ref[...])


def _new_h16(h_ref, a_ref, wc_ref, mask_ref):
  h = h_ref[...]
  s = (a_ref[0] + a_ref[1]) * mask_ref[...]
  cnt = (a_ref[0, :, K:K + 1] + a_ref[1, :, K:K + 1])
  return h + s + cnt * _dot(h, wc_ref[...])


def _update_body(h_ref, a_ref, wc_ref, wb_ref, mask_ref, hn_ref, g_ref):
  hn = _new_h16(h_ref, a_ref, wc_ref, mask_ref)
  hn_ref[...] = hn
  g_ref[...] = _dot(hn, wb_ref[...])


def _final_body(h_ref, a_ref, wc_ref, wo_ref, mask_ref, o_ref):
  hn = _new_h16(h_ref, a_ref, wc_ref, mask_ref)
  o_ref[...] = _dot(hn, wo_ref[...])


def _sc_step_body(ord_hbm, ei_dst, ei_src, g_hbm, zeros_hbm, out_hbm,
                  ord0, ord1, dst0, dst1, src0, src1, rows0, rows1,
                  so0, so1, si0, si1, sg0, sg1, ss0, ss1, accum):
  c = lax.axis_index("c")
  s = lax.axis_index("s")
  w = c * NS + s

  pltpu.sync_copy(zeros_hbm.at[pl.ds(s * RPT, RPT)],
                  accum.at[pl.ds(s * RPT, RPT)])
  plsc.subcore_barrier()

  nrows = NROW // NW + jnp.where(w < W_HI, 1, 0)
  row0 = (NROW // NW) * w + jnp.minimum(w, W_HI)
  pairs = nrows // 2

  def o_start(b, ordv, sem):
    pltpu.async_copy(ord_hbm.at[row0 + b], ordv, sem)

  def o_wait(ordv, sem):
    pltpu.make_async_copy(ord_hbm.at[0], ordv, sem).wait()

  def idx_start(ordv, dstv, srcv):
    pltpu.async_copy(ei_dst.at[ordv], dstv, si0)
    pltpu.async_copy(ei_src.at[ordv], srcv, si1)

  def idx_wait(dstv, srcv):
    pltpu.make_async_copy(ei_dst.at[dstv], dstv, si0).wait()
    pltpu.make_async_copy(ei_src.at[srcv], srcv, si1).wait()

  def g_start(srcv, rowsv, sem):
    pltpu.async_copy(g_hbm.at[srcv], rowsv, sem)

  def g_wait(srcv, rowsv, sem):
    pltpu.make_async_copy(g_hbm.at[srcv], rowsv, sem).wait()

  def s_start(rowsv, dstv, sem):
    pltpu.async_copy(rowsv, accum.at[dstv], sem, add=True)

  def s_wait(rowsv, dstv, sem):
    pltpu.make_async_copy(rowsv, accum.at[dstv], sem).wait()

  o_start(0, ord0, so0)
  o_start(1, ord1, so1)
  o_wait(ord0, so0)
  idx_start(ord0, dst0, src0)

  @pl.loop(0, pairs)
  def _(i):
    a = 2 * i
    b = a + 1
    idx_wait(dst0, src0)
    g_start(src0, rows0, sg0)

    @pl.when(a + 2 < nrows)
    def _():
      o_start(a + 2, ord0, so0)

    @pl.when(i > 0)
    def _():
      s_wait(rows1, dst1, ss1)

    o_wait(ord1, so1)
    idx_start(ord1, dst1, src1)
    g_wait(src0, rows0, sg0)
    s_start(rows0, dst0, ss0)
    idx_wait(dst1, src1)
    g_start(src1, rows1, sg1)

    @pl.when(b + 2 < nrows)
    def _():
      o_start(b + 2, ord1, so1)

    s_wait(rows0, dst0, ss0)

    @pl.when(a + 2 < nrows)
    def _():
      o_wait(ord0, so0)
      idx_start(ord0, dst0, src0)

    g_wait(src1, rows1, sg1)
    s_start(rows1, dst1, ss1)

  @pl.when(nrows % 2 == 1)
  def _():
    idx_wait(dst0, src0)
    g_start(src0, rows0, sg0)
    g_wait(src0, rows0, sg0)
    s_start(rows0, dst0, ss0)
    s_wait(rows0, dst0, ss0)

  s_wait(rows1, dst1, ss1)

  plsc.subcore_barrier()
  pltpu.sync_copy(accum.at[pl.ds(s * RPT, RPT)],
                  out_hbm.at[c].at[pl.ds(s * RPT, RPT)])


def kernel(x, edge_index, orders, W_mp, b_mp, W_out, b_out):
  f32 = jnp.float32
  i32 = jnp.int32
  ei = edge_index.astype(i32)
  ei_dst = ei[0]
  ei_src = ei[1]
  ords_p = orders.astype(i32).reshape(N_ORDERS, NROW, ROW)

  h16 = jnp.concatenate(
      [x, jnp.ones((N_NODES, 1), f32), jnp.zeros((N_NODES, GW - K - 1), f32)],
      axis=1)
  h16 = jnp.concatenate([h16, jnp.zeros((N_PAD - N_NODES, GW), f32)], axis=0)
  wa = W_mp[:, :K]
  wb = W_mp[:, K:]
  wb16 = jnp.zeros((GW, GW), f32).at[:K, :K].set(wb.T).at[K, K].set(1.0)
  wc16 = (jnp.zeros((GW, GW), f32).at[:K, :K].set((wa - wb).T)
          .at[K, :K].set(b_mp))
  wo16 = (jnp.zeros((GW, GW), f32).at[:K, :K].set(W_out.T)
          .at[K, :K].set(b_out))
  mask = jnp.zeros((1, GW), f32).at[0, :K].set(1.0)
  zeros_tbl = jnp.zeros((N_PAD, GW), f32)

  ngrid = N_PAD // BN
  _hs = pl.BlockSpec((BN, GW), lambda i: (i, 0))
  _as = pl.BlockSpec((NC, BN, GW), lambda i: (0, i, 0))
  _ws = pl.BlockSpec((GW, GW), lambda i: (0, 0))
  _ms = pl.BlockSpec((1, GW), lambda i: (0, 0))
  build = pl.pallas_call(
      _build_body, grid=(ngrid,), in_specs=[_hs, _ws], out_specs=_hs,
      out_shape=jax.ShapeDtypeStruct((N_PAD, GW), f32))
  update = pl.pallas_call(
      _update_body, grid=(ngrid,), in_specs=[_hs, _as, _ws, _ws, _ms],
      out_specs=(_hs, _hs),
      out_shape=(jax.ShapeDtypeStruct((N_PAD, GW), f32),
                 jax.ShapeDtypeStruct((N_PAD, GW), f32)))
  final = pl.pallas_call(
      _final_body, grid=(ngrid,), in_specs=[_hs, _as, _ws, _ws, _ms],
      out_specs=_hs,
      out_shape=jax.ShapeDtypeStruct((N_PAD, GW), f32))

  mesh = plsc.VectorSubcoreMesh(core_axis_name="c", subcore_axis_name="s")
  sc_step = functools.partial(
      pl.kernel,
      out_type=jax.ShapeDtypeStruct((NC, N_PAD, GW), f32),
      mesh=mesh,
      compiler_params=pltpu.CompilerParams(use_tc_tiling_on_sc=False),
      scratch_types=(
          [pltpu.VMEM((ROW,), i32) for _ in range(6)]
          + [pltpu.VMEM((ROW, GW), f32) for _ in range(2)]
          + [pltpu.SemaphoreType.DMA for _ in range(8)]
          + [pltpu.VMEM_SHARED((N_PAD, GW), f32)]
      ),
  )(_sc_step_body)

  h = h16
  g = build(h16, wb16)
  for i in range(N_ORDERS - 1):
    a = sc_step(ords_p[i], ei_dst, ei_src, g, zeros_tbl)
    h, g = update(h, a, wc16, wb16, mask)
  a = sc_step(ords_p[N_ORDERS - 1], ei_dst, ei_src, g, zeros_tbl)
  out = final(h, a, wc16, wo16, mask)
  return out[:N_NODES, :DIM_OUT]


# speedup vs baseline: 2.0432x; 2.0432x over previous
import functools

import jax
import jax.numpy as jnp
from jax import lax
from jax.experimental import pallas as pl
from jax.experimental.pallas import tpu as pltpu
from jax.experimental.pallas import tpu_sc as plsc

N_NODES = 100000
K = 10
DIM_OUT = 10
GW = 16
N_EDGES = 6400000
EPO = 1600000
N_ORDERS = 4

NC, NS = 2, 16
ROW = 512
NROW = EPO // ROW
NW = NC * NS
W_HI = NROW - (NROW // NW) * NW
N_PAD = 100096
RPT = N_PAD // NS
N8 = N_PAD // 8
BR = N8 // 4


_PREC = lax.Precision.HIGHEST


def _dot(a, b):
  return jnp.dot(a, b, preferred_element_type=jnp.float32, precision=_PREC)


def _build_body(h_ref, wb_ref, g_ref):
  g_ref[...] = _dot(h_ref[...], wb_ref[...])


def _new_h8(h_ref, a_ref, wc_ref, ec_ref, mask_ref):
  h = h_ref[...]
  s = a_ref[0] + a_ref[1]
  cnt = jnp.dot(s, ec_ref[...], preferred_element_type=jnp.float32)
  return h + s * mask_ref[...] + cnt * _dot(h, wc_ref[...])


def _update_body(h_ref, a_ref, wc_ref, wb_ref, ec_ref, mask_ref,
                 hn_ref, g_ref):
  hn = _new_h8(h_ref, a_ref, wc_ref, ec_ref, mask_ref)
  hn_ref[...] = hn
  g_ref[...] = _dot(hn, wb_ref[...])


def _final_body(h_ref, a_ref, wc_ref, wo_ref, ec_ref, mask_ref, o_ref):
  hn = _new_h8(h_ref, a_ref, wc_ref, ec_ref, mask_ref)
  o_ref[...] = _dot(hn, wo_ref[...])


def _sc_step_body(ord_hbm, ei_hbm, g_hbm, zeros_hbm, out_hbm,
                  ord0, ord1, dst0, dst1, src0, src1, rows0, rows1,
                  so0, so1, si0, si1, sg0, sg1, ss0, ss1, accum):
  ei_dst = ei_hbm.at[0]
  ei_src = ei_hbm.at[1]
  c = lax.axis_index("c")
  s = lax.axis_index("s")
  w = c * NS + s

  pltpu.sync_copy(zeros_hbm.at[pl.ds(s * RPT, RPT)],
                  accum.at[pl.ds(s * RPT, RPT)])
  plsc.subcore_barrier()

  nrows = NROW // NW + jnp.where(w < W_HI, 1, 0)
  row0 = (NROW // NW) * w + jnp.minimum(w, W_HI)
  pairs = nrows // 2

  def o_start(b, ordv, sem):
    pltpu.async_copy(ord_hbm.at[pl.ds((row0 + b) * ROW, ROW)], ordv, sem)

  def o_wait(ordv, sem):
    pltpu.make_async_copy(ord_hbm.at[pl.ds(0, ROW)], ordv, sem).wait()

  def idx_start(ordv, dstv, srcv):
    pltpu.async_copy(ei_dst.at[ordv], dstv, si0)
    pltpu.async_copy(ei_src.at[ordv], srcv, si1)

  def idx_wait(dstv, srcv):
    pltpu.make_async_copy(ei_dst.at[dstv], dstv, si0).wait()
    pltpu.make_async_copy(ei_src.at[srcv], srcv, si1).wait()

  def g_start(srcv, rowsv, sem):
    pltpu.async_copy(g_hbm.at[srcv], rowsv, sem)

  def g_wait(srcv, rowsv, sem):
    pltpu.make_async_copy(g_hbm.at[srcv], rowsv, sem).wait()

  def s_start(rowsv, dstv, sem):
    pltpu.async_copy(rowsv, accum.at[dstv], sem, add=True)

  def s_wait(rowsv, dstv, sem):
    pltpu.make_async_copy(rowsv, accum.at[dstv], sem).wait()

  o_start(0, ord0, so0)
  o_start(1, ord1, so1)
  o_wait(ord0, so0)
  idx_start(ord0, dst0, src0)

  @pl.loop(0, pairs)
  def _(i):
    a = 2 * i
    b = a + 1
    idx_wait(dst0, src0)
    g_start(src0, rows0, sg0)

    @pl.when(a + 2 < nrows)
    def _():
      o_start(a + 2, ord0, so0)

    @pl.when(i > 0)
    def _():
      s_wait(rows1, dst1, ss1)

    o_wait(ord1, so1)
    idx_start(ord1, dst1, src1)
    g_wait(src0, rows0, sg0)
    s_start(rows0, dst0, ss0)
    idx_wait(dst1, src1)
    g_start(src1, rows1, sg1)

    @pl.when(b + 2 < nrows)
    def _():
      o_start(b + 2, ord1, so1)

    s_wait(rows0, dst0, ss0)

    @pl.when(a + 2 < nrows)
    def _():
      o_wait(ord0, so0)
      idx_start(ord0, dst0, src0)

    g_wait(src1, rows1, sg1)
    s_start(rows1, dst1, ss1)

  @pl.when(nrows % 2 == 1)
  def _():
    idx_wait(dst0, src0)
    g_start(src0, rows0, sg0)
    g_wait(src0, rows0, sg0)
    s_start(rows0, dst0, ss0)
    s_wait(rows0, dst0, ss0)

  s_wait(rows1, dst1, ss1)

  plsc.subcore_barrier()
  pltpu.sync_copy(accum.at[pl.ds(s * RPT, RPT)],
                  out_hbm.at[c].at[pl.ds(s * RPT, RPT)])


def kernel(x, edge_index, orders, W_mp, b_mp, W_out, b_out):
  f32 = jnp.float32
  i32 = jnp.int32
  ei = edge_index.astype(i32)
  ords = orders.astype(i32)

  h16 = jnp.concatenate(
      [x, jnp.ones((N_NODES, 1), f32), jnp.zeros((N_NODES, GW - K - 1), f32)],
      axis=1)
  h16 = jnp.concatenate([h16, jnp.zeros((N_PAD - N_NODES, GW), f32)], axis=0)
  h8 = h16.reshape(N8, 128)

  wa = W_mp[:, :K]
  wb = W_mp[:, K:]
  wb16 = jnp.zeros((GW, GW), f32).at[:K, :K].set(wb.T).at[K, K].set(1.0)
  wc16 = (jnp.zeros((GW, GW), f32).at[:K, :K].set((wa - wb).T)
          .at[K, :K].set(b_mp))
  wo16 = (jnp.zeros((GW, GW), f32).at[:K, :K].set(W_out.T)
          .at[K, :K].set(b_out))
  cnt16 = jnp.zeros((GW, GW), f32).at[K, :].set(1.0)
  eye8 = jnp.eye(8, dtype=f32)
  wb_bd = jnp.kron(eye8, wb16)
  wc_bd = jnp.kron(eye8, wc16)
  wo_bd = jnp.kron(eye8, wo16)
  ec_bd = jnp.kron(eye8, cnt16)
  mask8 = jnp.tile(jnp.zeros((1, GW), f32).at[0, :K].set(1.0), (1, 8))
  zeros_tbl = jnp.zeros((N_PAD, GW), f32)

  ngrid = N8 // BR
  _hs = pl.BlockSpec((BR, 128), lambda i: (i, 0))
  _as = pl.BlockSpec((NC, BR, 128), lambda i: (0, i, 0))
  _ws = pl.BlockSpec((128, 128), lambda i: (0, 0))
  _ms = pl.BlockSpec((1, 128), lambda i: (0, 0))
  build = pl.pallas_call(
      _build_body, grid=(ngrid,), in_specs=[_hs, _ws], out_specs=_hs,
      out_shape=jax.ShapeDtypeStruct((N8, 128), f32))
  update = pl.pallas_call(
      _update_body, grid=(ngrid,), in_specs=[_hs, _as, _ws, _ws, _ws, _ms],
      out_specs=(_hs, _hs),
      out_shape=(jax.ShapeDtypeStruct((N8, 128), f32),
                 jax.ShapeDtypeStruct((N8, 128), f32)))
  final = pl.pallas_call(
      _final_body, grid=(ngrid,), in_specs=[_hs, _as, _ws, _ws, _ws, _ms],
      out_specs=_hs,
      out_shape=jax.ShapeDtypeStruct((N8, 128), f32))

  mesh = plsc.VectorSubcoreMesh(core_axis_name="c", subcore_axis_name="s")
  sc_step = functools.partial(
      pl.kernel,
      out_type=jax.ShapeDtypeStruct((NC, N_PAD, GW), f32),
      mesh=mesh,
      compiler_params=pltpu.CompilerParams(use_tc_tiling_on_sc=False),
      scratch_types=(
          [pltpu.VMEM((ROW,), i32) for _ in range(6)]
          + [pltpu.VMEM((ROW, GW), f32) for _ in range(2)]
          + [pltpu.SemaphoreType.DMA for _ in range(8)]
          + [pltpu.VMEM_SHARED((N_PAD, GW), f32)]
      ),
  )(_sc_step_body)

  h = h8
  g = build(h8, wb_bd)
  for i in range(N_ORDERS - 1):
    a = sc_step(ords[i], ei, g.reshape(N_PAD, GW), zeros_tbl)
    h, g = update(h, a.reshape(NC, N8, 128), wc_bd, wb_bd, ec_bd, mask8)
  a = sc_step(ords[N_ORDERS - 1], ei, g.reshape(N_PAD, GW), zeros_tbl)
  out = final(h, a.reshape(NC, N8, 128), wc_bd, wo_bd, ec_bd, mask8)
  return out.reshape(N_PAD, GW)[:N_NODES, :DIM_OUT]
